# baseline (speedup 1.0000x reference)
import functools

import jax
import jax.numpy as jnp
from jax import lax
from jax.experimental import pallas as pl
from jax.experimental.pallas import tpu as pltpu
from jax.experimental.pallas import tpu_sc as plsc

EMB_DIM = 32
BATCH = 16384
HIST = 50

NC, NS = 2, 16
NW = NC * NS
COLS_PER_W = BATCH // NW
NBT = COLS_PER_W // 128
UNITS = HIST * NBT
NBUF = 8
NG = UNITS // NBUF

_mesh = plsc.VectorSubcoreMesh(core_axis_name="c", subcore_axis_name="s")


@functools.partial(
    pl.kernel,
    mesh=_mesh,
    out_type=jax.ShapeDtypeStruct((HIST, EMB_DIM // 8, BATCH // 128, 8, 128),
                                  jnp.float32),
    compiler_params=pltpu.CompilerParams(use_tc_tiling_on_sc=False,
                                         needs_layout_passes=False),
    scratch_types=[
        pltpu.VMEM((HIST, COLS_PER_W), jnp.int32),
        pltpu.VMEM((NBUF, 128, EMB_DIM), jnp.float32),
        pltpu.VMEM((NBUF, 1, EMB_DIM // 8, 1, 8, 128), jnp.float32),
        pltpu.SemaphoreType.DMA((NBUF,)),
        pltpu.SemaphoreType.DMA((NBUF,)),
    ],
)
def _embed_gather(idx_hbm, table_hbm, out_hbm, idx_v, gbufs, tbufs,
                  sem_g, sem_o):
    wid = lax.axis_index("s") * NC + lax.axis_index("c")
    col0 = wid * COLS_PER_W
    pltpu.sync_copy(idx_hbm.at[:, pl.ds(col0, COLS_PER_W)], idx_v)

    iota16 = lax.iota(jnp.int32, 16)
    zeros16 = jnp.zeros((16,), jnp.int32)
    rows_k = [iota16 + 16 * k for k in range(8)]

    def start_gather(j, b):
        h = j // NBT
        bt = j % NBT
        pltpu.async_copy(
            table_hbm.at[idx_v.at[h, pl.ds(bt * 128, 128)]],
            gbufs.at[b], sem_g.at[b],
        )

    def wait_gather(b):
        pltpu.make_async_copy(
            table_hbm.at[idx_v.at[0, pl.ds(0, 128)]],
            gbufs.at[b], sem_g.at[b],
        ).wait()

    def transpose(b):
        for d in range(EMB_DIM):
            cols = zeros16 + d
            for k in range(8):
                v = plsc.load_gather(gbufs.at[b], [rows_k[k], cols])
                tbufs[b, 0, d // 8, 0, d % 8, pl.ds(16 * k, 16)] = v

    def start_outcopy(j, b):
        h = j // NBT
        bt = j % NBT
        pltpu.async_copy(
            tbufs.at[b],
            out_hbm.at[pl.ds(h, 1), pl.ds(0, EMB_DIM // 8),
                       pl.ds(wid * NBT + bt, 1), pl.ds(0, 8), pl.ds(0, 128)],
            sem_o.at[b],
        )

    def wait_outcopy(b):
        pltpu.make_async_copy(
            tbufs.at[b],
            out_hbm.at[pl.ds(0, 1), pl.ds(0, EMB_DIM // 8),
                       pl.ds(0, 1), pl.ds(0, 8), pl.ds(0, 128)],
            sem_o.at[b],
        ).wait()

    for b in range(NBUF):
        start_gather(b, b)

    def body(g, carry):
        for b in range(NBUF):
            j = g * NBUF + b
            wait_gather(b)

            @pl.when(g > 0)
            def _():
                wait_outcopy(b)

            transpose(b)
            start_outcopy(j, b)

            @pl.when(g < NG - 1)
            def _():
                start_gather(j + NBUF, b)

        return carry

    lax.fori_loop(0, NG, body, 0)

    for b in range(NBUF):
        wait_outcopy(b)


def kernel(sent_list, table):
    idx_lin = jnp.abs(sent_list.T)
    out5d = _embed_gather(idx_lin, table)
    return out5d.transpose((2, 4, 0, 1, 3)).reshape(BATCH, HIST, EMB_DIM)


# speedup vs baseline: 1.1267x; 1.1267x over previous
import functools

import jax
import jax.numpy as jnp
from jax import lax
from jax.experimental import pallas as pl
from jax.experimental.pallas import tpu as pltpu
from jax.experimental.pallas import tpu_sc as plsc

EMB_DIM = 32
BATCH = 16384
HIST = 50

NC, NS = 2, 16
NW = NC * NS
COLS_PER_W = BATCH // NW
NBT = COLS_PER_W // 128
NBUF = 2

_mesh = plsc.VectorSubcoreMesh(core_axis_name="c", subcore_axis_name="s")


@functools.partial(
    pl.kernel,
    mesh=_mesh,
    out_type=jax.ShapeDtypeStruct((HIST, EMB_DIM // 8, BATCH // 128, 8, 128),
                                  jnp.float32),
    compiler_params=pltpu.CompilerParams(use_tc_tiling_on_sc=False,
                                         needs_layout_passes=False),
    scratch_types=[
        pltpu.VMEM((HIST, COLS_PER_W), jnp.int32),
        pltpu.VMEM((NBUF, COLS_PER_W, EMB_DIM), jnp.float32),
        pltpu.VMEM((NBUF, 1, EMB_DIM // 8, NBT, 8, 128), jnp.float32),
        pltpu.SemaphoreType.DMA((NBUF,)),
        pltpu.SemaphoreType.DMA((NBUF,)),
    ],
)
def _embed_gather(idx_hbm, table_hbm, out_hbm, idx_v, gbufs, tbufs,
                  sem_g, sem_o):
    wid = lax.axis_index("s") * NC + lax.axis_index("c")
    col0 = wid * COLS_PER_W
    pltpu.sync_copy(idx_hbm.at[:, pl.ds(col0, COLS_PER_W)], idx_v)

    iota16 = lax.iota(jnp.int32, 16)
    zeros16 = jnp.zeros((16,), jnp.int32)
    rows_k = [iota16 + 16 * k for k in range(8)]

    def start_gather(h, b):
        pltpu.async_copy(
            table_hbm.at[idx_v.at[h]],
            gbufs.at[b], sem_g.at[b],
        )

    def wait_gather(b):
        pltpu.make_async_copy(
            table_hbm.at[idx_v.at[0]],
            gbufs.at[b], sem_g.at[b],
        ).wait()

    def transpose(b):
        def bt_body(bt, carry):
            r0 = bt * 128
            for d in range(EMB_DIM):
                cols = zeros16 + d
                for k in range(8):
                    v = plsc.load_gather(gbufs.at[b],
                                         [r0 + rows_k[k], cols])
                    tbufs[b, 0, d // 8, bt, d % 8, pl.ds(16 * k, 16)] = v
            return carry
        lax.fori_loop(0, NBT, bt_body, 0)

    def start_outcopy(h, b):
        pltpu.async_copy(
            tbufs.at[b],
            out_hbm.at[pl.ds(h, 1), pl.ds(0, EMB_DIM // 8),
                       pl.ds(wid * NBT, NBT), pl.ds(0, 8), pl.ds(0, 128)],
            sem_o.at[b],
        )

    def wait_outcopy(b):
        pltpu.make_async_copy(
            tbufs.at[b],
            out_hbm.at[pl.ds(0, 1), pl.ds(0, EMB_DIM // 8),
                       pl.ds(0, NBT), pl.ds(0, 8), pl.ds(0, 128)],
            sem_o.at[b],
        ).wait()

    for b in range(NBUF):
        start_gather(b, b)

    def body(g, carry):
        for b in range(NBUF):
            h = g * NBUF + b
            wait_gather(b)

            @pl.when(g > 0)
            def _():
                wait_outcopy(b)

            transpose(b)
            start_outcopy(h, b)

            @pl.when(g < HIST // NBUF - 1)
            def _():
                start_gather(h + NBUF, b)

        return carry

    lax.fori_loop(0, HIST // NBUF, body, 0)

    for b in range(NBUF):
        wait_outcopy(b)


def kernel(sent_list, table):
    idx_lin = jnp.abs(sent_list.T)
    out5d = _embed_gather(idx_lin, table)
    return out5d.transpose((2, 4, 0, 1, 3)).reshape(BATCH, HIST, EMB_DIM)


# speedup vs baseline: 1.1282x; 1.0014x over previous
import functools

import jax
import jax.numpy as jnp
from jax import lax
from jax.experimental import pallas as pl
from jax.experimental.pallas import tpu as pltpu
from jax.experimental.pallas import tpu_sc as plsc

EMB_DIM = 32
BATCH = 16384
HIST = 50

NC, NS = 2, 16
NW = NC * NS
COLS_PER_W = BATCH // NW
NBT = COLS_PER_W // 128
NGB = 3
NTB = 2

_mesh = plsc.VectorSubcoreMesh(core_axis_name="c", subcore_axis_name="s")


@functools.partial(
    pl.kernel,
    mesh=_mesh,
    out_type=jax.ShapeDtypeStruct((HIST, EMB_DIM // 8, BATCH // 128, 8, 128),
                                  jnp.float32),
    compiler_params=pltpu.CompilerParams(use_tc_tiling_on_sc=False,
                                         needs_layout_passes=False),
    scratch_types=[
        pltpu.VMEM((HIST, COLS_PER_W), jnp.int32),
        pltpu.VMEM((NGB, COLS_PER_W, EMB_DIM), jnp.float32),
        pltpu.VMEM((NTB, 1, EMB_DIM // 8, NBT, 8, 128), jnp.float32),
        pltpu.SemaphoreType.DMA((NGB,)),
        pltpu.SemaphoreType.DMA((NTB,)),
    ],
)
def _embed_gather(idx_hbm, table_hbm, out_hbm, idx_v, gbufs, tbufs,
                  sem_g, sem_o):
    wid = lax.axis_index("s") * NC + lax.axis_index("c")
    col0 = wid * COLS_PER_W
    pltpu.sync_copy(idx_hbm.at[:, pl.ds(col0, COLS_PER_W)], idx_v)

    iota16 = lax.iota(jnp.int32, 16)
    zeros16 = jnp.zeros((16,), jnp.int32)
    rows_k = [iota16 + 16 * k for k in range(8)]

    def start_gather(h, bg):
        pltpu.async_copy(
            table_hbm.at[idx_v.at[h]],
            gbufs.at[bg], sem_g.at[bg],
        )

    def wait_gather(bg):
        pltpu.make_async_copy(
            table_hbm.at[idx_v.at[0]],
            gbufs.at[bg], sem_g.at[bg],
        ).wait()

    def transpose(bg, bt_):
        def bt_body(bt, carry):
            r0 = bt * 128
            for d in range(EMB_DIM):
                cols = zeros16 + d
                for k in range(8):
                    v = plsc.load_gather(gbufs.at[bg],
                                         [r0 + rows_k[k], cols])
                    tbufs[bt_, 0, d // 8, bt, d % 8, pl.ds(16 * k, 16)] = v
            return carry
        lax.fori_loop(0, NBT, bt_body, 0)

    def start_outcopy(h, bt_):
        pltpu.async_copy(
            tbufs.at[bt_],
            out_hbm.at[pl.ds(h, 1), pl.ds(0, EMB_DIM // 8),
                       pl.ds(wid * NBT, NBT), pl.ds(0, 8), pl.ds(0, 128)],
            sem_o.at[bt_],
        )

    def wait_outcopy(bt_):
        pltpu.make_async_copy(
            tbufs.at[bt_],
            out_hbm.at[pl.ds(0, 1), pl.ds(0, EMB_DIM // 8),
                       pl.ds(0, NBT), pl.ds(0, 8), pl.ds(0, 128)],
            sem_o.at[bt_],
        ).wait()

    for j in range(NGB):
        start_gather(j, j)

    def body(j, carry):
        bg = j % NGB
        bt_ = j % NTB
        wait_gather(bg)

        @pl.when(j >= NTB)
        def _():
            wait_outcopy(bt_)

        transpose(bg, bt_)
        start_outcopy(j, bt_)

        @pl.when(j < HIST - NGB)
        def _():
            start_gather(j + NGB, bg)

        return carry

    lax.fori_loop(0, HIST, body, 0)

    for bt_ in range(NTB):
        wait_outcopy(bt_)


def kernel(sent_list, table):
    idx_lin = jnp.abs(sent_list.T)
    out5d = _embed_gather(idx_lin, table)
    return out5d.transpose((2, 4, 0, 1, 3)).reshape(BATCH, HIST, EMB_DIM)


# speedup vs baseline: 1.1284x; 1.0002x over previous
import functools

import jax
import jax.numpy as jnp
from jax import lax
from jax.experimental import pallas as pl
from jax.experimental.pallas import tpu as pltpu
from jax.experimental.pallas import tpu_sc as plsc

EMB_DIM = 32
BATCH = 16384
HIST = 50

NC, NS = 2, 16
NW = NC * NS
COLS_PER_W = BATCH // NW
NBT = COLS_PER_W // 128
NGB = 3
NTB = 3

_mesh = plsc.VectorSubcoreMesh(core_axis_name="c", subcore_axis_name="s")


@functools.partial(
    pl.kernel,
    mesh=_mesh,
    out_type=jax.ShapeDtypeStruct((HIST, EMB_DIM // 8, BATCH // 128, 8, 128),
                                  jnp.float32),
    compiler_params=pltpu.CompilerParams(use_tc_tiling_on_sc=False,
                                         needs_layout_passes=False),
    scratch_types=[
        pltpu.VMEM((HIST, COLS_PER_W), jnp.int32),
        pltpu.VMEM((NGB, COLS_PER_W, EMB_DIM), jnp.float32),
        pltpu.VMEM((NTB, 1, EMB_DIM // 8, NBT, 8, 128), jnp.float32),
        pltpu.SemaphoreType.DMA((NGB,)),
        pltpu.SemaphoreType.DMA((NTB,)),
    ],
)
def _embed_gather(idx_hbm, table_hbm, out_hbm, idx_v, gbufs, tbufs,
                  sem_g, sem_o):
    wid = lax.axis_index("s") * NC + lax.axis_index("c")
    col0 = wid * COLS_PER_W
    pltpu.sync_copy(idx_hbm.at[:, pl.ds(col0, COLS_PER_W)], idx_v)

    iota16 = lax.iota(jnp.int32, 16)
    zeros16 = jnp.zeros((16,), jnp.int32)
    rows_k = [iota16 + 16 * k for k in range(8)]

    def start_gather(h, bg):
        pltpu.async_copy(
            table_hbm.at[idx_v.at[h]],
            gbufs.at[bg], sem_g.at[bg],
        )

    def wait_gather(bg):
        pltpu.make_async_copy(
            table_hbm.at[idx_v.at[0]],
            gbufs.at[bg], sem_g.at[bg],
        ).wait()

    def transpose(bg, bt_):
        def bt_body(bt, carry):
            r0 = bt * 128
            for d in range(EMB_DIM):
                cols = zeros16 + d
                for k in range(8):
                    v = plsc.load_gather(gbufs.at[bg],
                                         [r0 + rows_k[k], cols])
                    tbufs[bt_, 0, d // 8, bt, d % 8, pl.ds(16 * k, 16)] = v
            return carry
        lax.fori_loop(0, NBT, bt_body, 0)

    def start_outcopy(h, bt_):
        pltpu.async_copy(
            tbufs.at[bt_],
            out_hbm.at[pl.ds(h, 1), pl.ds(0, EMB_DIM // 8),
                       pl.ds(wid * NBT, NBT), pl.ds(0, 8), pl.ds(0, 128)],
            sem_o.at[bt_],
        )

    def wait_outcopy(bt_):
        pltpu.make_async_copy(
            tbufs.at[bt_],
            out_hbm.at[pl.ds(0, 1), pl.ds(0, EMB_DIM // 8),
                       pl.ds(0, NBT), pl.ds(0, 8), pl.ds(0, 128)],
            sem_o.at[bt_],
        ).wait()

    for j in range(NGB):
        start_gather(j, j)

    def body(j, carry):
        bg = j % NGB
        bt_ = j % NTB
        wait_gather(bg)

        @pl.when(j >= NTB)
        def _():
            wait_outcopy(bt_)

        transpose(bg, bt_)
        start_outcopy(j, bt_)

        @pl.when(j < HIST - NGB)
        def _():
            start_gather(j + NGB, bg)

        return carry

    lax.fori_loop(0, HIST, body, 0)

    for bt_ in range(NTB):
        wait_outcopy(bt_)


def kernel(sent_list, table):
    idx_lin = jnp.abs(sent_list.T)
    out5d = _embed_gather(idx_lin, table)
    return out5d.transpose((2, 4, 0, 1, 3)).reshape(BATCH, HIST, EMB_DIM)
